# Initial kernel scaffold; baseline (speedup 1.0000x reference)
import jax
import jax.numpy as jnp
from jax.experimental import pallas as pl


def kernel(z, codebook):
    raise NotImplementedError("write your pallas kernel here")


# speedup vs baseline: 1.0097x; 1.0097x over previous
import functools

import jax
import jax.numpy as jnp
from jax import lax
from jax.experimental import pallas as pl
from jax.experimental.pallas import tpu as pltpu
from jax.experimental.pallas import tpu_sc as plsc

_N_E = 8192
_DIM = 32
_BETA = 0.25
_ROWS = 8 * 32 * 32

_R = 256
_NB = _ROWS // _R

_NC, _NS, _L = 2, 16, 16
_NW = _NC * _NS
_BPW = _ROWS // _NW
_KCH = 128
_NCH = _BPW // _KCH


def _tc_argmin_body(z_ref, cb_ref, idx_ref):
    zb = z_ref[...]
    cb = cb_ref[...]
    m = lax.dot_general(zb, cb, (((1,), (1,)), ((), ())),
                        preferred_element_type=jnp.float32)
    zsq = jnp.sum(zb * zb, axis=1, keepdims=True)
    csq = jnp.sum(cb * cb, axis=1)
    d = (zsq + csq[None, :]) - 2.0 * m
    idx = jnp.argmin(d, axis=1).astype(jnp.int32)
    idx_ref[0, 0, :] = idx


def _tc_argmin(z_flat, codebook):
    idx3 = pl.pallas_call(
        _tc_argmin_body,
        grid=(_NB,),
        in_specs=[
            pl.BlockSpec((_R, _DIM), lambda i: (i, 0)),
            pl.BlockSpec((_N_E, _DIM), lambda i: (0, 0)),
        ],
        out_specs=pl.BlockSpec((1, 1, _R), lambda i: (i, 0, 0)),
        out_shape=jax.ShapeDtypeStruct((_NB, 1, _R), jnp.int32),
    )(z_flat, codebook)
    return idx3.reshape(_ROWS)


_CBPAD = 128


def _sc_lookup_call(codebook_pad, idx_w, z_flat):
    @functools.partial(
        pl.kernel,
        mesh=plsc.VectorSubcoreMesh(core_axis_name="c", subcore_axis_name="s"),
        out_type=[
            jax.ShapeDtypeStruct((_ROWS, _DIM), jnp.float32),
            jax.ShapeDtypeStruct((_NW, _L), jnp.float32),
        ],
        scratch_types=[
            pltpu.VMEM((_NCH, _KCH), jnp.int32),
            pltpu.VMEM((_BPW, _CBPAD), jnp.float32),
            pltpu.VMEM((_BPW, _DIM), jnp.float32),
            pltpu.VMEM((_BPW, _DIM), jnp.float32),
            pltpu.VMEM((1, _L), jnp.float32),
            pltpu.SemaphoreType.DMA,
        ],
    )
    def _sc_lookup(cb_hbm, idx_hbm, z_hbm, zq_hbm, part_hbm,
                   idx_v, rows_v, zq_c, z_v, acc_v, sem):
        wid = lax.axis_index("s") * _NC + lax.axis_index("c")
        base = wid * _BPW
        pltpu.sync_copy(idx_hbm.at[wid], idx_v)
        pltpu.sync_copy(z_hbm.at[pl.ds(base, _BPW)], z_v)
        copies = [
            pltpu.async_copy(cb_hbm.at[idx_v.at[j]],
                             rows_v.at[pl.ds(j * _KCH, _KCH)], sem)
            for j in range(_NCH)
        ]
        for c in copies:
            c.wait()

        def body(i, acc):
            r0 = rows_v[i, pl.ds(0, _L)]
            r1 = rows_v[i, pl.ds(_L, _L)]
            zq_c[i, pl.ds(0, _L)] = r0
            zq_c[i, pl.ds(_L, _L)] = r1
            d0 = r0 - z_v[i, pl.ds(0, _L)]
            d1 = r1 - z_v[i, pl.ds(_L, _L)]
            return acc + (d0 * d0 + d1 * d1)

        acc = lax.fori_loop(0, _BPW, body, jnp.zeros((_L,), jnp.float32))
        acc_v[0, :] = acc
        pltpu.sync_copy(zq_c, zq_hbm.at[pl.ds(base, _BPW)])
        pltpu.sync_copy(acc_v, part_hbm.at[pl.ds(wid, 1)])

    return _sc_lookup(codebook_pad, idx_w, z_flat)


def kernel(z, codebook):
    z_p = jnp.transpose(z, (0, 2, 3, 1))
    z_flat = z_p.reshape(_ROWS, _DIM)
    idx = _tc_argmin(z_flat, codebook)
    idx_w = idx.reshape(_NW, _NCH, _KCH)
    cb_pad = jnp.pad(codebook, ((0, 0), (0, _CBPAD - _DIM)))
    zq_flat, part = _sc_lookup_call(cb_pad, idx_w, z_flat)
    mean_sq = jnp.sum(part) / jnp.float32(_ROWS * _DIM)
    loss = mean_sq + _BETA * mean_sq
    zq_st = z_flat + (zq_flat - z_flat)
    z_q_out = jnp.transpose(zq_st.reshape(z_p.shape), (0, 3, 1, 2))
    return (loss, z_q_out)
